# baseline (speedup 1.0000x reference)
import functools

import jax
import jax.numpy as jnp
from jax import lax
from jax.experimental import pallas as pl
from jax.experimental.pallas import tpu as pltpu
from jax.experimental.pallas import tpu_sc as plsc

N_NODES = 10000
N_EDGES = 320000
H_NODE = 128
H_EDGE = 16
D_OUT = 2 * H_NODE
EPS = 1e-5

NC = 2
NS = 16
NW = NC * NS
E_PER_W = N_EDGES // NW
CH = 80
N_CHUNKS = E_PER_W // CH
G_SLOTS = 5
ACC_ROWS = 10240
ROWS_PER_TILE = ACC_ROWS // NS

_MESH = plsc.VectorSubcoreMesh(
    core_axis_name="c", subcore_axis_name="s", num_cores=NC, num_subcores=NS
)


G_CH = 40
TBL_CH = 640


def _make_gather(n_edges):
    e_per_w = n_edges // NW
    n_chunks = e_per_w // G_CH
    assert n_chunks % G_SLOTS == 0 and e_per_w % 8 == 0

    @functools.partial(
        pl.kernel,
        out_type=jax.ShapeDtypeStruct((n_edges * H_NODE // 2,), jnp.int32),
        mesh=_MESH,
        scratch_types=[
            pltpu.VMEM((e_per_w,), jnp.int32),
            [pltpu.VMEM((G_CH, H_NODE), jnp.float32) for _ in range(G_SLOTS)],
            [pltpu.VMEM((G_CH * H_NODE // 2,), jnp.int32) for _ in range(G_SLOTS)],
            [pltpu.SemaphoreType.DMA for _ in range(G_SLOTS)],
            [pltpu.SemaphoreType.DMA for _ in range(G_SLOTS)],
            pltpu.VMEM_SHARED((N_NODES, H_NODE), jnp.float32),
        ],
    )
    def gather(node_hbm, idx_hbm, out_hbm, idx_all, rows, rows16, gsems, ssems, tbl_sh):
        sid = lax.axis_index("s")
        wid = sid * NC + lax.axis_index("c")
        base = wid * e_per_w

        t0 = sid * TBL_CH

        @pl.when(sid < NS - 1)
        def _stage_full():
            pltpu.sync_copy(
                node_hbm.at[pl.ds(t0, TBL_CH)], tbl_sh.at[pl.ds(t0, TBL_CH)]
            )

        @pl.when(sid == NS - 1)
        def _stage_last():
            last0 = (NS - 1) * TBL_CH
            nlast = N_NODES - last0
            pltpu.sync_copy(
                node_hbm.at[pl.ds(last0, nlast)], tbl_sh.at[pl.ds(last0, nlast)]
            )

        pltpu.sync_copy(idx_hbm.at[pl.ds(base, e_per_w)], idx_all)
        plsc.subcore_barrier()

        def fire(c, s):
            pltpu.async_copy(
                tbl_sh.at[idx_all.at[pl.ds(c * G_CH, G_CH)]], rows[s], gsems[s]
            )

        def wait_gather(s):
            pltpu.make_async_copy(node_hbm.at[pl.ds(0, G_CH)], rows[s], gsems[s]).wait()

        def pack16(s):
            half = jnp.full((16,), 0x8000, jnp.int32)
            hmask = jnp.full((16,), -65536, jnp.int32)

            def prow(r, carry):
                for blk in range(H_NODE // 32):
                    a = lax.bitcast_convert_type(
                        rows[s][r, pl.ds(32 * blk, 16)], jnp.int32
                    )
                    b = lax.bitcast_convert_type(
                        rows[s][r, pl.ds(32 * blk + 16, 16)], jnp.int32
                    )
                    lo = lax.shift_right_logical(a + half, 16)
                    hi = (b + half) & hmask
                    rows16[s][pl.ds(r * (H_NODE // 2) + 16 * blk, 16)] = lo | hi
                return carry

            lax.fori_loop(0, G_CH, prow, 0)

        def store(c, s):
            pltpu.async_copy(
                rows16[s],
                out_hbm.at[
                    pl.ds((base + c * G_CH) * (H_NODE // 2), G_CH * H_NODE // 2)
                ],
                ssems[s],
            )

        def wait_store(s):
            pltpu.make_async_copy(
                rows16[s], out_hbm.at[pl.ds(0, G_CH * H_NODE // 2)], ssems[s]
            ).wait()

        fire(0, 0)
        fire(1, 1)
        fire(2, 2)

        def body(j, carry):
            for d in range(G_SLOTS):
                c = j * G_SLOTS + d
                wait_gather(d)
                pack16(d)
                store(c, d)
                cn = c + 3
                sn = (d + 3) % G_SLOTS

                @pl.when(cn < n_chunks)
                def _():
                    @pl.when(c >= 2)
                    def _():
                        wait_store(sn)

                    fire(cn, sn)

            return carry

        lax.fori_loop(0, n_chunks // G_SLOTS, body, 0)
        for s in range(G_SLOTS):
            wait_store(s)

    return gather


SC_SLOTS = 4


def _make_scatter(n_edges):
    e_per_w = n_edges // NW
    n_chunks = e_per_w // CH
    assert e_per_w % CH == 0

    @functools.partial(
        pl.kernel,
        out_type=jax.ShapeDtypeStruct((NC, N_NODES, H_NODE), jnp.float32),
        mesh=_MESH,
        scratch_types=[
            [pltpu.VMEM((CH,), jnp.int32) for _ in range(SC_SLOTS)],
            [pltpu.VMEM((CH, H_NODE), jnp.float32) for _ in range(SC_SLOTS)],
            [pltpu.SemaphoreType.DMA for _ in range(SC_SLOTS)],
            [pltpu.SemaphoreType.DMA for _ in range(SC_SLOTS)],
            [pltpu.SemaphoreType.DMA for _ in range(SC_SLOTS)],
            pltpu.VMEM_SHARED((ACC_ROWS, H_NODE), jnp.float32),
        ],
    )
    def scatter(msg_hbm, idx_hbm, out_hbm, idxb, rowsb, isems, lsems, asems, acc_sh):
        cid = lax.axis_index("c")
        sid = lax.axis_index("s")
        wid = cid * NS + sid
        base = wid * e_per_w

        def zrow(r, carry):
            for cc in range(H_NODE // 16):
                rowsb[0][r, pl.ds(cc * 16, 16)] = jnp.zeros((16,), jnp.float32)
            return carry

        lax.fori_loop(0, CH, zrow, 0)
        r0 = sid * ROWS_PER_TILE
        for ofs in range(0, ROWS_PER_TILE, CH):
            pltpu.sync_copy(rowsb[0], acc_sh.at[pl.ds(r0 + ofs, CH)])
        plsc.subcore_barrier()

        def fire_load(c, s):
            ebase = base + c * CH
            pltpu.async_copy(idx_hbm.at[pl.ds(ebase, CH)], idxb[s], isems[s])
            pltpu.async_copy(msg_hbm.at[pl.ds(ebase, CH)], rowsb[s], lsems[s])

        def wait_load(s):
            pltpu.make_async_copy(idx_hbm.at[pl.ds(0, CH)], idxb[s], isems[s]).wait()
            pltpu.make_async_copy(msg_hbm.at[pl.ds(0, CH)], rowsb[s], lsems[s]).wait()

        def fire_add(s):
            pltpu.async_copy(rowsb[s], acc_sh.at[idxb[s]], asems[s], add=True)

        def wait_add(s):
            pltpu.make_async_copy(rowsb[s], acc_sh.at[idxb[s]], asems[s]).wait()

        fire_load(0, 0)
        fire_load(1, 1)

        def step(j, carry):
            for d in range(SC_SLOTS):
                c = j * SC_SLOTS + d

                @pl.when(c < n_chunks)
                def _():
                    wait_load(d)
                    fire_add(d)
                    cn = c + 2
                    sn = (d + 2) % SC_SLOTS

                    @pl.when(cn < n_chunks)
                    def _():
                        @pl.when(c >= 2)
                        def _():
                            wait_add(sn)

                        fire_load(cn, sn)

            return carry

        lax.fori_loop(0, (n_chunks + SC_SLOTS - 1) // SC_SLOTS, step, 0)
        for s in range(SC_SLOTS):
            wait_add(s)
        plsc.subcore_barrier()

        @pl.when(sid < NS - 1)
        def _copy_full():
            pltpu.sync_copy(
                acc_sh.at[pl.ds(r0, ROWS_PER_TILE)],
                out_hbm.at[cid, pl.ds(r0, ROWS_PER_TILE)],
            )

        @pl.when(sid == NS - 1)
        def _copy_last():
            last0 = (NS - 1) * ROWS_PER_TILE
            nlast = N_NODES - last0
            pltpu.sync_copy(
                acc_sh.at[pl.ds(last0, nlast)], out_hbm.at[cid, pl.ds(last0, nlast)]
            )

    return scatter


R_BLK = 6400
E_A = 128000
E_B = N_EDGES - E_A


def _tc_stats_body(g_ref, e_ref, wn_ref, we_ref, b_ref, out_ref):
    c = (
        jnp.dot(
            g_ref[...],
            wn_ref[...],
            preferred_element_type=jnp.float32,
        )
        + jnp.dot(
            e_ref[...].astype(jnp.bfloat16),
            we_ref[...],
            preferred_element_type=jnp.float32,
        )
        + b_ref[...]
    )
    s = jnp.sum(c, axis=0, keepdims=True)
    s2 = jnp.sum(c * c, axis=0, keepdims=True)
    blk = jnp.concatenate([s, s2], axis=0)

    @pl.when(pl.program_id(0) == 0)
    def _init():
        out_ref[...] = blk

    @pl.when(pl.program_id(0) > 0)
    def _acc():
        out_ref[...] += blk


def _tc_msg_body(g_ref, e_ref, wn_ref, we_ref, b_ref, st_ref, w1_ref, b1_ref, out_ref):
    c = (
        jnp.dot(
            g_ref[...],
            wn_ref[...],
            preferred_element_type=jnp.float32,
        )
        + jnp.dot(
            e_ref[...].astype(jnp.bfloat16),
            we_ref[...],
            preferred_element_type=jnp.float32,
        )
        + b_ref[...]
    )
    mean = st_ref[0:1, :] * (1.0 / N_EDGES)
    var = st_ref[1:2, :] * (1.0 / N_EDGES) - mean * mean
    inv = lax.rsqrt(var + EPS)
    scale = w1_ref[...] * inv
    shift = b1_ref[...] - mean * scale
    y = c * scale + shift
    out_ref[...] = jax.nn.sigmoid(y[:, :H_NODE]) * jnp.tanh(y[:, H_NODE:])


def _make_stats(n_edges):
    n_blks = n_edges // R_BLK
    return pl.pallas_call(
        _tc_stats_body,
        grid=(n_blks,),
        in_specs=[
            pl.BlockSpec((R_BLK, H_NODE), lambda j: (j, 0)),
            pl.BlockSpec((R_BLK, H_EDGE), lambda j: (j, 0)),
            pl.BlockSpec((H_NODE, D_OUT), lambda j: (0, 0)),
            pl.BlockSpec((H_EDGE, D_OUT), lambda j: (0, 0)),
            pl.BlockSpec((1, D_OUT), lambda j: (0, 0)),
        ],
        out_specs=pl.BlockSpec((2, D_OUT), lambda j: (0, 0)),
        out_shape=jax.ShapeDtypeStruct((2, D_OUT), jnp.float32),
    )


def _make_msg(n_edges):
    n_blks = n_edges // R_BLK
    return pl.pallas_call(
        _tc_msg_body,
        grid=(n_blks,),
        in_specs=[
            pl.BlockSpec((R_BLK, H_NODE), lambda j: (j, 0)),
            pl.BlockSpec((R_BLK, H_EDGE), lambda j: (j, 0)),
            pl.BlockSpec((H_NODE, D_OUT), lambda j: (0, 0)),
            pl.BlockSpec((H_EDGE, D_OUT), lambda j: (0, 0)),
            pl.BlockSpec((1, D_OUT), lambda j: (0, 0)),
            pl.BlockSpec((2, D_OUT), lambda j: (0, 0)),
            pl.BlockSpec((1, D_OUT), lambda j: (0, 0)),
            pl.BlockSpec((1, D_OUT), lambda j: (0, 0)),
        ],
        out_specs=pl.BlockSpec((R_BLK, H_NODE), lambda j: (j, 0)),
        out_shape=jax.ShapeDtypeStruct((n_edges, H_NODE), jnp.float32),
    )


def _tc_final_body(agg_a_ref, agg_b_ref, node_ref, w_ref, b_ref, out_ref):
    agg = agg_a_ref[0] + agg_a_ref[1] + agg_b_ref[0] + agg_b_ref[1]
    mean = jnp.mean(agg, axis=0, keepdims=True)
    var = jnp.mean((agg - mean) ** 2, axis=0, keepdims=True)
    y = (agg - mean) * lax.rsqrt(var + EPS) * w_ref[...] + b_ref[...]
    out_ref[...] = jnp.tanh(node_ref[...] + y)


_tc_final = pl.pallas_call(
    _tc_final_body,
    out_shape=jax.ShapeDtypeStruct((N_NODES, H_NODE), jnp.float32),
)

_gather_a = _make_gather(E_A)
_gather_b = _make_gather(E_B)
_scatter_a = _make_scatter(E_A)
_scatter_b = _make_scatter(E_B)
_stats_a = _make_stats(E_A)
_stats_b = _make_stats(E_B)
_msg_a = _make_msg(E_A)
_msg_b = _make_msg(E_B)


def kernel(node_emb, edge_emb, i, lin_W, lin_b, bn1_w, bn1_b, bn_w, bn_b):
    idx = i.astype(jnp.int32)
    idx_a, idx_b = idx[:E_A], idx[E_A:]
    e_a, e_b = edge_emb[:E_A], edge_emb[E_A:]
    perm = [
        32 * blk + (16 * (tt % 2) + tt // 2)
        for blk in range(H_NODE // 32)
        for tt in range(32)
    ]
    wn = lin_W[:, :H_NODE].T[jnp.array(perm), :].astype(jnp.bfloat16)
    we = lin_W[:, H_NODE:].T.astype(jnp.bfloat16)
    b2 = lin_b.reshape(1, D_OUT)
    w1 = bn1_w.reshape(1, D_OUT)
    b1 = bn1_b.reshape(1, D_OUT)
    wb = bn_w.reshape(1, H_NODE)
    bb = bn_b.reshape(1, H_NODE)

    g_a = lax.bitcast_convert_type(
        _gather_a(node_emb, idx_a), jnp.bfloat16
    ).reshape(E_A, H_NODE)
    g_b = lax.bitcast_convert_type(
        _gather_b(node_emb, idx_b), jnp.bfloat16
    ).reshape(E_B, H_NODE)
    st_a = _stats_a(g_a, e_a, wn, we, b2)
    st_b = _stats_b(g_b, e_b, wn, we, b2)
    st = st_a + st_b
    msg_a = _msg_a(g_a, e_a, wn, we, b2, st, w1, b1)
    agg_a = _scatter_a(msg_a, idx_a)
    msg_b = _msg_b(g_b, e_b, wn, we, b2, st, w1, b1)
    agg_b = _scatter_b(msg_b, idx_b)
    return _tc_final(agg_a, agg_b, node_emb, wb, bb)


# speedup vs baseline: 2.5659x; 2.5659x over previous
import functools

import jax
import jax.numpy as jnp
from jax import lax
from jax.experimental import pallas as pl
from jax.experimental.pallas import tpu as pltpu
from jax.experimental.pallas import tpu_sc as plsc

N_NODES = 10000
N_EDGES = 320000
H_NODE = 128
H_EDGE = 16
D_OUT = 2 * H_NODE
EPS = 1e-5

NC = 2
NS = 16
NW = NC * NS
E_PER_W = N_EDGES // NW
CH = 80
N_CHUNKS = E_PER_W // CH
G_SLOTS = 5
G_RING = 4
ACC_ROWS = 10240
ROWS_PER_TILE = ACC_ROWS // NS

_MESH = plsc.VectorSubcoreMesh(
    core_axis_name="c", subcore_axis_name="s", num_cores=NC, num_subcores=NS
)


G_CH = 80
TBL_CH = 640


def _make_gather(n_edges):
    e_per_w = n_edges // NW
    n_chunks = e_per_w // G_CH
    assert e_per_w % G_CH == 0 and e_per_w % 8 == 0

    @functools.partial(
        pl.kernel,
        out_type=jax.ShapeDtypeStruct((n_edges, H_NODE), jnp.float32),
        mesh=_MESH,
        scratch_types=[
            pltpu.VMEM((e_per_w,), jnp.int32),
            [pltpu.VMEM((G_CH, H_NODE), jnp.float32) for _ in range(G_RING)],
            [pltpu.SemaphoreType.DMA for _ in range(G_RING)],
            [pltpu.SemaphoreType.DMA for _ in range(G_RING)],
            pltpu.VMEM_SHARED((N_NODES, H_NODE), jnp.float32),
        ],
    )
    def gather(node_hbm, idx_hbm, out_hbm, idx_all, rows, gsems, ssems, tbl_sh):
        sid = lax.axis_index("s")
        wid = sid * NC + lax.axis_index("c")
        base = wid * e_per_w

        t0 = sid * TBL_CH

        @pl.when(sid < NS - 1)
        def _stage_full():
            pltpu.sync_copy(
                node_hbm.at[pl.ds(t0, TBL_CH)], tbl_sh.at[pl.ds(t0, TBL_CH)]
            )

        @pl.when(sid == NS - 1)
        def _stage_last():
            last0 = (NS - 1) * TBL_CH
            nlast = N_NODES - last0
            pltpu.sync_copy(
                node_hbm.at[pl.ds(last0, nlast)], tbl_sh.at[pl.ds(last0, nlast)]
            )

        pltpu.sync_copy(idx_hbm.at[pl.ds(base, e_per_w)], idx_all)
        plsc.subcore_barrier()

        def fire(c, s):
            pltpu.async_copy(
                tbl_sh.at[idx_all.at[pl.ds(c * G_CH, G_CH)]], rows[s], gsems[s]
            )

        def wait_gather(s):
            pltpu.make_async_copy(out_hbm.at[pl.ds(0, G_CH)], rows[s], gsems[s]).wait()

        def store(c, s):
            pltpu.async_copy(
                rows[s], out_hbm.at[pl.ds(base + c * G_CH, G_CH)], ssems[s]
            )

        def wait_store(s):
            pltpu.make_async_copy(rows[s], out_hbm.at[pl.ds(0, G_CH)], ssems[s]).wait()

        fire(0, 0)
        fire(1, 1)

        def body(j, carry):
            for d in range(G_RING):
                c = j * G_RING + d

                @pl.when(c < n_chunks)
                def _():
                    wait_gather(d)
                    store(c, d)
                    cn = c + 2
                    sn = (d + 2) % G_RING

                    @pl.when(cn < n_chunks)
                    def _():
                        @pl.when(c >= 2)
                        def _():
                            wait_store(sn)

                        fire(cn, sn)

            return carry

        lax.fori_loop(0, (n_chunks + G_RING - 1) // G_RING, body, 0)
        for s in range(G_RING):
            wait_store(s)

    return gather


SC_SLOTS = 4


def _make_scatter(n_edges):
    e_per_w = n_edges // NW
    n_chunks = e_per_w // CH
    assert e_per_w % CH == 0

    @functools.partial(
        pl.kernel,
        out_type=jax.ShapeDtypeStruct((NC, N_NODES, H_NODE), jnp.float32),
        mesh=_MESH,
        scratch_types=[
            [pltpu.VMEM((CH,), jnp.int32) for _ in range(SC_SLOTS)],
            [pltpu.VMEM((CH, H_NODE), jnp.float32) for _ in range(SC_SLOTS)],
            [pltpu.SemaphoreType.DMA for _ in range(SC_SLOTS)],
            [pltpu.SemaphoreType.DMA for _ in range(SC_SLOTS)],
            [pltpu.SemaphoreType.DMA for _ in range(SC_SLOTS)],
            pltpu.VMEM_SHARED((ACC_ROWS, H_NODE), jnp.float32),
        ],
    )
    def scatter(msg_hbm, idx_hbm, out_hbm, idxb, rowsb, isems, lsems, asems, acc_sh):
        cid = lax.axis_index("c")
        sid = lax.axis_index("s")
        wid = cid * NS + sid
        base = wid * e_per_w

        def zrow(r, carry):
            for cc in range(H_NODE // 16):
                rowsb[0][r, pl.ds(cc * 16, 16)] = jnp.zeros((16,), jnp.float32)
            return carry

        lax.fori_loop(0, CH, zrow, 0)
        r0 = sid * ROWS_PER_TILE
        for ofs in range(0, ROWS_PER_TILE, CH):
            pltpu.sync_copy(rowsb[0], acc_sh.at[pl.ds(r0 + ofs, CH)])
        plsc.subcore_barrier()

        def fire_load(c, s):
            ebase = base + c * CH
            pltpu.async_copy(idx_hbm.at[pl.ds(ebase, CH)], idxb[s], isems[s])
            pltpu.async_copy(msg_hbm.at[pl.ds(ebase, CH)], rowsb[s], lsems[s])

        def wait_load(s):
            pltpu.make_async_copy(idx_hbm.at[pl.ds(0, CH)], idxb[s], isems[s]).wait()
            pltpu.make_async_copy(msg_hbm.at[pl.ds(0, CH)], rowsb[s], lsems[s]).wait()

        def fire_add(s):
            pltpu.async_copy(rowsb[s], acc_sh.at[idxb[s]], asems[s], add=True)

        def wait_add(s):
            pltpu.make_async_copy(rowsb[s], acc_sh.at[idxb[s]], asems[s]).wait()

        fire_load(0, 0)
        fire_load(1, 1)

        def step(j, carry):
            for d in range(SC_SLOTS):
                c = j * SC_SLOTS + d

                @pl.when(c < n_chunks)
                def _():
                    wait_load(d)
                    fire_add(d)
                    cn = c + 2
                    sn = (d + 2) % SC_SLOTS

                    @pl.when(cn < n_chunks)
                    def _():
                        @pl.when(c >= 2)
                        def _():
                            wait_add(sn)

                        fire_load(cn, sn)

            return carry

        lax.fori_loop(0, (n_chunks + SC_SLOTS - 1) // SC_SLOTS, step, 0)
        for s in range(SC_SLOTS):
            wait_add(s)
        plsc.subcore_barrier()

        @pl.when(sid < NS - 1)
        def _copy_full():
            pltpu.sync_copy(
                acc_sh.at[pl.ds(r0, ROWS_PER_TILE)],
                out_hbm.at[cid, pl.ds(r0, ROWS_PER_TILE)],
            )

        @pl.when(sid == NS - 1)
        def _copy_last():
            last0 = (NS - 1) * ROWS_PER_TILE
            nlast = N_NODES - last0
            pltpu.sync_copy(
                acc_sh.at[pl.ds(last0, nlast)], out_hbm.at[cid, pl.ds(last0, nlast)]
            )

    return scatter


R_BLK = 6400
E_A = 128000
E_B = N_EDGES - E_A


def _tc_stats_body(g_ref, e_ref, wn_ref, we_ref, b_ref, out_ref):
    c = (
        jnp.dot(
            g_ref[...].astype(jnp.bfloat16),
            wn_ref[...],
            preferred_element_type=jnp.float32,
        )
        + jnp.dot(
            e_ref[...].astype(jnp.bfloat16),
            we_ref[...],
            preferred_element_type=jnp.float32,
        )
        + b_ref[...]
    )
    s = jnp.sum(c, axis=0, keepdims=True)
    s2 = jnp.sum(c * c, axis=0, keepdims=True)
    blk = jnp.concatenate([s, s2], axis=0)

    @pl.when(pl.program_id(0) == 0)
    def _init():
        out_ref[...] = blk

    @pl.when(pl.program_id(0) > 0)
    def _acc():
        out_ref[...] += blk


def _tc_msg_body(g_ref, e_ref, wn_ref, we_ref, b_ref, st_ref, w1_ref, b1_ref, out_ref):
    c = (
        jnp.dot(
            g_ref[...].astype(jnp.bfloat16),
            wn_ref[...],
            preferred_element_type=jnp.float32,
        )
        + jnp.dot(
            e_ref[...].astype(jnp.bfloat16),
            we_ref[...],
            preferred_element_type=jnp.float32,
        )
        + b_ref[...]
    )
    mean = st_ref[0:1, :] * (1.0 / N_EDGES)
    var = st_ref[1:2, :] * (1.0 / N_EDGES) - mean * mean
    inv = lax.rsqrt(var + EPS)
    scale = w1_ref[...] * inv
    shift = b1_ref[...] - mean * scale
    y = c * scale + shift
    out_ref[...] = jax.nn.sigmoid(y[:, :H_NODE]) * jnp.tanh(y[:, H_NODE:])


def _make_stats(n_edges):
    n_blks = n_edges // R_BLK
    return pl.pallas_call(
        _tc_stats_body,
        grid=(n_blks,),
        in_specs=[
            pl.BlockSpec((R_BLK, H_NODE), lambda j: (j, 0)),
            pl.BlockSpec((R_BLK, H_EDGE), lambda j: (j, 0)),
            pl.BlockSpec((H_NODE, D_OUT), lambda j: (0, 0)),
            pl.BlockSpec((H_EDGE, D_OUT), lambda j: (0, 0)),
            pl.BlockSpec((1, D_OUT), lambda j: (0, 0)),
        ],
        out_specs=pl.BlockSpec((2, D_OUT), lambda j: (0, 0)),
        out_shape=jax.ShapeDtypeStruct((2, D_OUT), jnp.float32),
    )


def _make_msg(n_edges):
    n_blks = n_edges // R_BLK
    return pl.pallas_call(
        _tc_msg_body,
        grid=(n_blks,),
        in_specs=[
            pl.BlockSpec((R_BLK, H_NODE), lambda j: (j, 0)),
            pl.BlockSpec((R_BLK, H_EDGE), lambda j: (j, 0)),
            pl.BlockSpec((H_NODE, D_OUT), lambda j: (0, 0)),
            pl.BlockSpec((H_EDGE, D_OUT), lambda j: (0, 0)),
            pl.BlockSpec((1, D_OUT), lambda j: (0, 0)),
            pl.BlockSpec((2, D_OUT), lambda j: (0, 0)),
            pl.BlockSpec((1, D_OUT), lambda j: (0, 0)),
            pl.BlockSpec((1, D_OUT), lambda j: (0, 0)),
        ],
        out_specs=pl.BlockSpec((R_BLK, H_NODE), lambda j: (j, 0)),
        out_shape=jax.ShapeDtypeStruct((n_edges, H_NODE), jnp.float32),
    )


def _tc_final_body(agg_a_ref, agg_b_ref, node_ref, w_ref, b_ref, out_ref):
    agg = agg_a_ref[0] + agg_a_ref[1] + agg_b_ref[0] + agg_b_ref[1]
    mean = jnp.mean(agg, axis=0, keepdims=True)
    var = jnp.mean((agg - mean) ** 2, axis=0, keepdims=True)
    y = (agg - mean) * lax.rsqrt(var + EPS) * w_ref[...] + b_ref[...]
    out_ref[...] = jnp.tanh(node_ref[...] + y)


_tc_final = pl.pallas_call(
    _tc_final_body,
    out_shape=jax.ShapeDtypeStruct((N_NODES, H_NODE), jnp.float32),
)

_gather_a = _make_gather(E_A)
_gather_b = _make_gather(E_B)
_scatter_a = _make_scatter(E_A)
_scatter_b = _make_scatter(E_B)
_stats_a = _make_stats(E_A)
_stats_b = _make_stats(E_B)
_msg_a = _make_msg(E_A)
_msg_b = _make_msg(E_B)


def kernel(node_emb, edge_emb, i, lin_W, lin_b, bn1_w, bn1_b, bn_w, bn_b):
    idx = i.astype(jnp.int32)
    idx_a, idx_b = idx[:E_A], idx[E_A:]
    e_a, e_b = edge_emb[:E_A], edge_emb[E_A:]
    wn = lin_W[:, :H_NODE].T.astype(jnp.bfloat16)
    we = lin_W[:, H_NODE:].T.astype(jnp.bfloat16)
    b2 = lin_b.reshape(1, D_OUT)
    w1 = bn1_w.reshape(1, D_OUT)
    b1 = bn1_b.reshape(1, D_OUT)
    wb = bn_w.reshape(1, H_NODE)
    bb = bn_b.reshape(1, H_NODE)

    g_a = _gather_a(node_emb, idx_a)
    g_b = _gather_b(node_emb, idx_b)
    st_a = _stats_a(g_a, e_a, wn, we, b2)
    st_b = _stats_b(g_b, e_b, wn, we, b2)
    st = st_a + st_b
    msg_a = _msg_a(g_a, e_a, wn, we, b2, st, w1, b1)
    agg_a = _scatter_a(msg_a, idx_a)
    msg_b = _msg_b(g_b, e_b, wn, we, b2, st, w1, b1)
    agg_b = _scatter_b(msg_b, idx_b)
    return _tc_final(agg_a, agg_b, node_emb, wb, bb)
